# Initial kernel scaffold; baseline (speedup 1.0000x reference)
import jax
import jax.numpy as jnp
from jax.experimental import pallas as pl


def kernel(x, edge_attr, edge_index, node_tables, edge_tables, W_enc, W_dec):
    raise NotImplementedError("write your pallas kernel here")


# speedup vs baseline: 4.2329x; 4.2329x over previous
import functools

import jax
import jax.numpy as jnp
from jax import lax
from jax.experimental import pallas as pl
from jax.experimental.pallas import tpu as pltpu
from jax.experimental.pallas import tpu_sc as plsc

N = 100000
E = 1600000
NODE_FIELDS = 8
EDGE_FIELDS = 4
D = 32
H = 64

NC = 2
NS = 16
SUB = 128

HALF = N // NC
TRASH = HALF
ACC_ROWS = 50176
NODE_FULL = HALF // SUB
NODE_TAIL = HALF - NODE_FULL * SUB
NODE_TAIL_BASE = NODE_FULL * SUB
ESUB = E // SUB
ZROWS = 32
ZSUB = ACC_ROWS // ZROWS


def _sc_embed_aggregate(node_tables, edge_tables, xT, attrT, dst):
  mesh = plsc.VectorSubcoreMesh(
      core_axis_name="c", subcore_axis_name="s", num_cores=NC, num_subcores=NS)

  @functools.partial(
      pl.kernel,
      out_type=jax.ShapeDtypeStruct((N, D), jnp.float32),
      mesh=mesh,
      compiler_params=pltpu.CompilerParams(use_tc_tiling_on_sc=False),
      scratch_types=[
          pltpu.VMEM_SHARED((ACC_ROWS, D), jnp.float32),
          pltpu.VMEM((ZROWS, D), jnp.float32),
          pltpu.VMEM((NODE_FIELDS, SUB), jnp.int32),
          pltpu.VMEM((EDGE_FIELDS, SUB), jnp.int32),
          pltpu.VMEM((SUB,), jnp.int32),
          pltpu.VMEM((SUB,), jnp.int32),
          pltpu.VMEM((SUB,), jnp.int32),
          pltpu.VMEM((NODE_TAIL,), jnp.int32),
          pltpu.VMEM((EDGE_FIELDS, SUB, D), jnp.float32),
          pltpu.SemaphoreType.DMA,
      ],
  )
  def k(node_tbl, edge_tbl, xT_h, attrT_h, dst_h, out, acc, zbuf, xbuf, abuf,
        dbuf, dloc, lin, lin_tail, rows, sem):
    c = lax.axis_index("c")
    s = lax.axis_index("s")
    nbase = c * HALF
    iota = lax.iota(jnp.int32, 16)
    zero16 = jnp.zeros((16,), jnp.float32)

    def zrow(i, _):
      zbuf[i, pl.ds(0, 16)] = zero16
      zbuf[i, pl.ds(16, 16)] = zero16
      return 0
    lax.fori_loop(0, ZROWS, zrow, 0)

    nz = (ZSUB - s + NS - 1) // NS
    def zbody(i, _):
      j = s + i * NS
      pltpu.sync_copy(zbuf, acc.at[pl.ds(j * ZROWS, ZROWS)])
      return 0
    lax.fori_loop(0, nz, zbody, 0)
    plsc.subcore_barrier()

    nn = (NODE_FULL - s + NS - 1) // NS
    def nbody(i, _):
      j = s + i * NS
      gbase = nbase + j * SUB
      cps = [pltpu.async_copy(xT_h.at[pl.ds(f * N + gbase, SUB)], xbuf.at[f], sem)
             for f in range(NODE_FIELDS)]
      for cp in cps:
        cp.wait()
      lbase = j * SUB
      for kk in range(SUB // 16):
        lin[pl.ds(kk * 16, 16)] = lbase + kk * 16 + iota
      for g in range(NODE_FIELDS // EDGE_FIELDS):
        fs = range(g * EDGE_FIELDS, (g + 1) * EDGE_FIELDS)
        cps = [pltpu.async_copy(node_tbl.at[f].at[xbuf.at[f]],
                                rows.at[f - g * EDGE_FIELDS], sem)
               for f in fs]
        for cp in cps:
          cp.wait()
        for f in fs:
          pltpu.sync_copy(rows.at[f - g * EDGE_FIELDS], acc.at[lin], add=True)
      return 0
    lax.fori_loop(0, nn, nbody, 0)

    @pl.when(s == 0)
    def _node_tail():
      for kk in range(NODE_TAIL // 16):
        lin_tail[pl.ds(kk * 16, 16)] = NODE_TAIL_BASE + kk * 16 + iota
      gbase = nbase + NODE_TAIL_BASE
      cps = [pltpu.async_copy(xT_h.at[pl.ds(f * N + gbase, NODE_TAIL)],
                              xbuf.at[f, pl.ds(0, NODE_TAIL)], sem)
             for f in range(NODE_FIELDS)]
      for cp in cps:
        cp.wait()
      for g in range(NODE_FIELDS // EDGE_FIELDS):
        fs = range(g * EDGE_FIELDS, (g + 1) * EDGE_FIELDS)
        cps = [pltpu.async_copy(node_tbl.at[f].at[xbuf.at[f, pl.ds(0, NODE_TAIL)]],
                                rows.at[f - g * EDGE_FIELDS, pl.ds(0, NODE_TAIL)],
                                sem)
               for f in fs]
        for cp in cps:
          cp.wait()
        for f in fs:
          pltpu.sync_copy(rows.at[f - g * EDGE_FIELDS, pl.ds(0, NODE_TAIL)],
                          acc.at[lin_tail], add=True)

    ne = (ESUB - s + NS - 1) // NS
    def ebody(i, _):
      j = s + i * NS
      ebase = j * SUB
      cps = [pltpu.async_copy(attrT_h.at[pl.ds(f * E + ebase, SUB)], abuf.at[f], sem)
             for f in range(EDGE_FIELDS)]
      cps.append(pltpu.async_copy(dst_h.at[pl.ds(ebase, SUB)], dbuf, sem))
      for cp in cps:
        cp.wait()
      for kk in range(SUB // 16):
        v = dbuf[pl.ds(kk * 16, 16)]
        loc = v - nbase
        ok = (v >= nbase) & (loc < HALF)
        dloc[pl.ds(kk * 16, 16)] = jnp.where(ok, loc, TRASH)
      cps = [pltpu.async_copy(edge_tbl.at[f].at[abuf.at[f]], rows.at[f], sem)
             for f in range(EDGE_FIELDS)]
      for cp in cps:
        cp.wait()
      for f in range(EDGE_FIELDS):
        pltpu.sync_copy(rows.at[f], acc.at[dloc], add=True)
      return 0
    lax.fori_loop(0, ne, ebody, 0)

    plsc.subcore_barrier()

    def xbody(i, _):
      j = s + i * NS
      pltpu.sync_copy(acc.at[pl.ds(j * SUB, SUB)],
                      out.at[pl.ds(nbase + j * SUB, SUB)])
      return 0
    lax.fori_loop(0, nn, xbody, 0)

    @pl.when(s == 0)
    def _exp_tail():
      pltpu.sync_copy(acc.at[pl.ds(NODE_TAIL_BASE, NODE_TAIL)],
                      out.at[pl.ds(nbase + NODE_TAIL_BASE, NODE_TAIL)])

  return k(node_tables, edge_tables, xT, attrT, dst)


MLP_BLK = 2000


def _mlp(agg, W_enc, W_dec):
  def body(a_ref, we_ref, wd_ref, o_ref):
    h = jnp.maximum(
        jnp.dot(a_ref[...], we_ref[...], preferred_element_type=jnp.float32),
        0.0)
    o_ref[...] = jnp.dot(h, wd_ref[...], preferred_element_type=jnp.float32)

  return pl.pallas_call(
      body,
      grid=(N // MLP_BLK,),
      in_specs=[
          pl.BlockSpec((MLP_BLK, D), lambda i: (i, 0)),
          pl.BlockSpec((D, H), lambda i: (0, 0)),
          pl.BlockSpec((H, D), lambda i: (0, 0)),
      ],
      out_specs=pl.BlockSpec((MLP_BLK, D), lambda i: (i, 0)),
      out_shape=jax.ShapeDtypeStruct((N, D), jnp.float32),
  )(agg, W_enc, W_dec)


def kernel(x, edge_attr, edge_index, node_tables, edge_tables, W_enc, W_dec):
  xT = jnp.transpose(x).reshape(-1)
  attrT = jnp.transpose(edge_attr).reshape(-1)
  dst = edge_index[1]
  agg = _sc_embed_aggregate(node_tables, edge_tables, xT, attrT, dst)
  return _mlp(agg, W_enc, W_dec)


# speedup vs baseline: 7.6238x; 1.8011x over previous
import functools

import jax
import jax.numpy as jnp
from jax import lax
from jax.experimental import pallas as pl
from jax.experimental.pallas import tpu as pltpu
from jax.experimental.pallas import tpu_sc as plsc

N = 100000
E = 1600000
NODE_FIELDS = 8
EDGE_FIELDS = 4
D = 32
H = 64

NC = 2
NS = 16
SUB = 128

HALF = N // NC
TRASH = HALF
ACC_ROWS = 50176
NODE_FULL = HALF // SUB
NODE_TAIL = HALF - NODE_FULL * SUB
NODE_TAIL_BASE = NODE_FULL * SUB
ESUB = E // SUB
ZROWS = 128
ZSUB = ACC_ROWS // ZROWS


def _sc_embed_aggregate(node_tables, edge_tables, xT, attrT, dst):
  mesh = plsc.VectorSubcoreMesh(
      core_axis_name="c", subcore_axis_name="s", num_cores=NC, num_subcores=NS)

  @functools.partial(
      pl.kernel,
      out_type=jax.ShapeDtypeStruct((N, D), jnp.float32),
      mesh=mesh,
      compiler_params=pltpu.CompilerParams(use_tc_tiling_on_sc=False),
      scratch_types=[
          pltpu.VMEM_SHARED((ACC_ROWS, D), jnp.float32),
          pltpu.VMEM((ZROWS, D), jnp.float32),
          pltpu.VMEM((2, NODE_FIELDS, SUB), jnp.int32),
          pltpu.VMEM((2, EDGE_FIELDS, SUB), jnp.int32),
          pltpu.VMEM((2, SUB), jnp.int32),
          pltpu.VMEM((2, SUB), jnp.int32),
          pltpu.VMEM((2, SUB), jnp.int32),
          pltpu.VMEM((NODE_TAIL,), jnp.int32),
          pltpu.VMEM((EDGE_FIELDS, SUB, D), jnp.float32),
          pltpu.SemaphoreType.DMA,
          pltpu.SemaphoreType.DMA,
          pltpu.SemaphoreType.DMA,
      ],
  )
  def k(node_tbl, edge_tbl, xT_h, attrT_h, dst_h, out, acc, zbuf, xbuf, abuf,
        dbuf, dloc, lin, lin_tail, rows, sem_in, sem_g, sem_s):
    c = lax.axis_index("c")
    s = lax.axis_index("s")
    nbase = c * HALF
    iota = lax.iota(jnp.int32, 16)
    zero16 = jnp.zeros((16,), jnp.float32)

    def zrow(i, _):
      zbuf[i, pl.ds(0, 16)] = zero16
      zbuf[i, pl.ds(16, 16)] = zero16
      return 0
    lax.fori_loop(0, ZROWS, zrow, 0)

    nz = (ZSUB - s + NS - 1) // NS
    def zbody(i, _):
      j = s + i * NS
      pltpu.async_copy(zbuf, acc.at[pl.ds(j * ZROWS, ZROWS)], sem_s)
      return 0
    lax.fori_loop(0, nz, zbody, 0)
    def zdrain(i, _):
      pltpu.make_async_copy(zbuf, acc.at[pl.ds(s * ZROWS, ZROWS)], sem_s).wait()
      return 0
    lax.fori_loop(0, nz, zdrain, 0)
    plsc.subcore_barrier()

    nn = (NODE_FULL - s + NS - 1) // NS

    def fire_inputs_n(j, b):
      for f in range(NODE_FIELDS):
        pltpu.async_copy(xT_h.at[pl.ds(f * N + nbase + j * SUB, SUB)],
                         xbuf.at[b, f], sem_in)

    fire_inputs_n(s, 0)

    def nbody(i, _):
      j = s + i * NS
      b = i & 1
      for f in range(NODE_FIELDS):
        pltpu.make_async_copy(xT_h.at[pl.ds(f * N + nbase + j * SUB, SUB)],
                              xbuf.at[b, f], sem_in).wait()

      @pl.when(i + 1 < nn)
      def _prefetch():
        fire_inputs_n(j + NS, 1 - b)

      lbase = j * SUB
      for kk in range(SUB // 16):
        lin[b, pl.ds(kk * 16, 16)] = lbase + kk * 16 + iota

      for w in range(NODE_FIELDS // 2):
        pA = 2 * (w & 1)
        planes = (pA, pA + 1)
        fields = (2 * w, 2 * w + 1)
        if w < 2:
          @pl.when(i > 0)
          def _drain(planes=planes):
            for p in planes:
              pltpu.make_async_copy(rows.at[p], acc.at[lin.at[1 - b]],
                                    sem_s).wait()
        else:
          for p in planes:
            pltpu.make_async_copy(rows.at[p], acc.at[lin.at[b]], sem_s).wait()
        gcps = [pltpu.async_copy(node_tbl.at[f].at[xbuf.at[b, f]],
                                 rows.at[p], sem_g)
                for f, p in zip(fields, planes)]
        for cp in gcps:
          cp.wait()
        for p in planes:
          pltpu.async_copy(rows.at[p], acc.at[lin.at[b]], sem_s, add=True)
      return 0
    lax.fori_loop(0, nn, nbody, 0)

    blast_n = (nn - 1) & 1
    for p in range(EDGE_FIELDS):
      pltpu.make_async_copy(rows.at[p], acc.at[lin.at[blast_n]], sem_s).wait()

    @pl.when(s == 0)
    def _node_tail():
      for kk in range(NODE_TAIL // 16):
        lin_tail[pl.ds(kk * 16, 16)] = NODE_TAIL_BASE + kk * 16 + iota
      gbase = nbase + NODE_TAIL_BASE
      cps = [pltpu.async_copy(xT_h.at[pl.ds(f * N + gbase, NODE_TAIL)],
                              xbuf.at[0, f, pl.ds(0, NODE_TAIL)], sem_in)
             for f in range(NODE_FIELDS)]
      for cp in cps:
        cp.wait()
      for g in range(2):
        fs = range(g * 4, g * 4 + 4)
        cps = [pltpu.async_copy(
            node_tbl.at[f].at[xbuf.at[0, f, pl.ds(0, NODE_TAIL)]],
            rows.at[f - g * 4, pl.ds(0, NODE_TAIL)], sem_g) for f in fs]
        for cp in cps:
          cp.wait()
        for f in fs:
          pltpu.sync_copy(rows.at[f - g * 4, pl.ds(0, NODE_TAIL)],
                          acc.at[lin_tail], add=True)

    ne = (ESUB - s + NS - 1) // NS

    def fire_inputs_e(j, b):
      for f in range(EDGE_FIELDS):
        pltpu.async_copy(attrT_h.at[pl.ds(f * E + j * SUB, SUB)],
                         abuf.at[b, f], sem_in)
      pltpu.async_copy(dst_h.at[pl.ds(j * SUB, SUB)], dbuf.at[b], sem_in)

    fire_inputs_e(s, 0)

    def ebody(i, _):
      j = s + i * NS
      b = i & 1
      for f in range(EDGE_FIELDS):
        pltpu.make_async_copy(attrT_h.at[pl.ds(f * E + j * SUB, SUB)],
                              abuf.at[b, f], sem_in).wait()
      pltpu.make_async_copy(dst_h.at[pl.ds(j * SUB, SUB)], dbuf.at[b],
                            sem_in).wait()

      @pl.when(i + 1 < ne)
      def _prefetch():
        fire_inputs_e(j + NS, 1 - b)

      for kk in range(SUB // 16):
        v = dbuf[b, pl.ds(kk * 16, 16)]
        loc = v - nbase
        ok = (v >= nbase) & (loc < HALF)
        dloc[b, pl.ds(kk * 16, 16)] = jnp.where(ok, loc, TRASH + kk * 16 + iota)

      for w in range(2):
        planes = (2 * w, 2 * w + 1)
        @pl.when(i > 0)
        def _drain(planes=planes):
          for p in planes:
            pltpu.make_async_copy(rows.at[p], acc.at[dloc.at[1 - b]],
                                  sem_s).wait()
        gcps = [pltpu.async_copy(edge_tbl.at[p].at[abuf.at[b, p]],
                                 rows.at[p], sem_g) for p in planes]
        for cp in gcps:
          cp.wait()
        for p in planes:
          pltpu.async_copy(rows.at[p], acc.at[dloc.at[b]], sem_s, add=True)
      return 0
    lax.fori_loop(0, ne, ebody, 0)

    blast_e = (ne - 1) & 1
    for p in range(EDGE_FIELDS):
      pltpu.make_async_copy(rows.at[p], acc.at[dloc.at[blast_e]], sem_s).wait()

    plsc.subcore_barrier()

    def xbody(i, _):
      j = s + i * NS
      pltpu.async_copy(acc.at[pl.ds(j * SUB, SUB)],
                       out.at[pl.ds(nbase + j * SUB, SUB)], sem_s)
      return 0
    lax.fori_loop(0, nn, xbody, 0)
    def xdrain(i, _):
      pltpu.make_async_copy(acc.at[pl.ds(s * SUB, SUB)],
                            out.at[pl.ds(nbase + s * SUB, SUB)], sem_s).wait()
      return 0
    lax.fori_loop(0, nn, xdrain, 0)

    @pl.when(s == 0)
    def _exp_tail():
      pltpu.sync_copy(acc.at[pl.ds(NODE_TAIL_BASE, NODE_TAIL)],
                      out.at[pl.ds(nbase + NODE_TAIL_BASE, NODE_TAIL)])

  return k(node_tables, edge_tables, xT, attrT, dst)


MLP_BLK = 2000


def _mlp(agg, W_enc, W_dec):
  def body(a_ref, we_ref, wd_ref, o_ref):
    h = jnp.maximum(
        jnp.dot(a_ref[...], we_ref[...], preferred_element_type=jnp.float32),
        0.0)
    o_ref[...] = jnp.dot(h, wd_ref[...], preferred_element_type=jnp.float32)

  return pl.pallas_call(
      body,
      grid=(N // MLP_BLK,),
      in_specs=[
          pl.BlockSpec((MLP_BLK, D), lambda i: (i, 0)),
          pl.BlockSpec((D, H), lambda i: (0, 0)),
          pl.BlockSpec((H, D), lambda i: (0, 0)),
      ],
      out_specs=pl.BlockSpec((MLP_BLK, D), lambda i: (i, 0)),
      out_shape=jax.ShapeDtypeStruct((N, D), jnp.float32),
  )(agg, W_enc, W_dec)


def kernel(x, edge_attr, edge_index, node_tables, edge_tables, W_enc, W_dec):
  xT = jnp.transpose(x).reshape(-1)
  attrT = jnp.transpose(edge_attr).reshape(-1)
  dst = edge_index[1]
  agg = _sc_embed_aggregate(node_tables, edge_tables, xT, attrT, dst)
  return _mlp(agg, W_enc, W_dec)


# speedup vs baseline: 7.7095x; 1.0112x over previous
import functools

import jax
import jax.numpy as jnp
from jax import lax
from jax.experimental import pallas as pl
from jax.experimental.pallas import tpu as pltpu
from jax.experimental.pallas import tpu_sc as plsc

N = 100000
E = 1600000
NODE_FIELDS = 8
EDGE_FIELDS = 4
D = 32
H = 64

NC = 2
NS = 16
SUB = 128

HALF = N // NC
TRASH = HALF
ACC_ROWS = 50176
NODE_FULL = HALF // SUB
NODE_TAIL = HALF - NODE_FULL * SUB
NODE_TAIL_BASE = NODE_FULL * SUB
ESUB = E // SUB
ZROWS = 128
ZSUB = ACC_ROWS // ZROWS


def _sc_embed_aggregate(node_tables, edge_tables, xT, attrT, ei_flat):
  mesh = plsc.VectorSubcoreMesh(
      core_axis_name="c", subcore_axis_name="s", num_cores=NC, num_subcores=NS)

  @functools.partial(
      pl.kernel,
      out_type=jax.ShapeDtypeStruct((N, D), jnp.float32),
      mesh=mesh,
      compiler_params=pltpu.CompilerParams(use_tc_tiling_on_sc=False),
      scratch_types=[
          pltpu.VMEM_SHARED((ACC_ROWS, D), jnp.float32),
          pltpu.VMEM((ZROWS, D), jnp.float32),
          pltpu.VMEM((2, NODE_FIELDS, SUB), jnp.int32),
          pltpu.VMEM((2, EDGE_FIELDS, SUB), jnp.int32),
          pltpu.VMEM((2, SUB), jnp.int32),
          pltpu.VMEM((2, SUB), jnp.int32),
          pltpu.VMEM((2, SUB), jnp.int32),
          pltpu.VMEM((NODE_TAIL,), jnp.int32),
          pltpu.VMEM((EDGE_FIELDS, SUB, D), jnp.float32),
          pltpu.SemaphoreType.DMA,
          pltpu.SemaphoreType.DMA,
          pltpu.SemaphoreType.DMA,
      ],
  )
  def k(node_tbl, edge_tbl, xT_h, attrT_h, ei_h, out, acc, zbuf, xbuf, abuf,
        dbuf, dloc, lin, lin_tail, rows, sem_in, sem_g, sem_s):
    c = lax.axis_index("c")
    s = lax.axis_index("s")
    nbase = c * HALF
    iota = lax.iota(jnp.int32, 16)
    zero16 = jnp.zeros((16,), jnp.float32)

    def zrow(i, _):
      zbuf[i, pl.ds(0, 16)] = zero16
      zbuf[i, pl.ds(16, 16)] = zero16
      return 0
    lax.fori_loop(0, ZROWS, zrow, 0)

    nz = (ZSUB - s + NS - 1) // NS
    def zbody(i, _):
      j = s + i * NS
      pltpu.async_copy(zbuf, acc.at[pl.ds(j * ZROWS, ZROWS)], sem_s)
      return 0
    lax.fori_loop(0, nz, zbody, 0)
    def zdrain(i, _):
      pltpu.make_async_copy(zbuf, acc.at[pl.ds(s * ZROWS, ZROWS)], sem_s).wait()
      return 0
    lax.fori_loop(0, nz, zdrain, 0)
    plsc.subcore_barrier()

    nn = (NODE_FULL - s + NS - 1) // NS

    def fire_inputs_n(j, b):
      for f in range(NODE_FIELDS):
        pltpu.async_copy(xT_h.at[pl.ds(f * N + nbase + j * SUB, SUB)],
                         xbuf.at[b, f], sem_in)

    fire_inputs_n(s, 0)

    def nbody(i, _):
      j = s + i * NS
      b = i & 1
      p0, p1 = 2 * b, 2 * b + 1
      for f in range(NODE_FIELDS):
        pltpu.make_async_copy(xT_h.at[pl.ds(f * N + nbase + j * SUB, SUB)],
                              xbuf.at[b, f], sem_in).wait()

      @pl.when(i + 1 < nn)
      def _prefetch():
        fire_inputs_n(j + NS, 1 - b)

      @pl.when(i >= 2)
      def _drain():
        for p in (p0, p1):
          pltpu.make_async_copy(rows.at[p], acc.at[lin.at[b]], sem_s).wait()

      lbase = j * SUB
      for kk in range(SUB // 16):
        lin[b, pl.ds(kk * 16, 16)] = lbase + kk * 16 + iota

      for w in range(NODE_FIELDS // 2):
        add = w > 0
        gcps = [pltpu.async_copy(node_tbl.at[2 * w].at[xbuf.at[b, 2 * w]],
                                 rows.at[p0], sem_g, add=add),
                pltpu.async_copy(node_tbl.at[2 * w + 1].at[xbuf.at[b, 2 * w + 1]],
                                 rows.at[p1], sem_g, add=add)]
        for cp in gcps:
          cp.wait()
      for p in (p0, p1):
        pltpu.async_copy(rows.at[p], acc.at[lin.at[b]], sem_s, add=True)
      return 0
    lax.fori_loop(0, nn, nbody, 0)

    blast_n = (nn - 1) & 1
    for bb in (blast_n, 1 - blast_n):
      for p in (2 * bb, 2 * bb + 1):
        pltpu.make_async_copy(rows.at[p], acc.at[lin.at[bb]], sem_s).wait()

    @pl.when(s == 0)
    def _node_tail():
      for kk in range(NODE_TAIL // 16):
        lin_tail[pl.ds(kk * 16, 16)] = NODE_TAIL_BASE + kk * 16 + iota
      gbase = nbase + NODE_TAIL_BASE
      cps = [pltpu.async_copy(xT_h.at[pl.ds(f * N + gbase, NODE_TAIL)],
                              xbuf.at[0, f, pl.ds(0, NODE_TAIL)], sem_in)
             for f in range(NODE_FIELDS)]
      for cp in cps:
        cp.wait()
      for w in range(NODE_FIELDS // 2):
        gcps = [pltpu.async_copy(
            node_tbl.at[2 * w + q].at[xbuf.at[0, 2 * w + q, pl.ds(0, NODE_TAIL)]],
            rows.at[q, pl.ds(0, NODE_TAIL)], sem_g, add=w > 0) for q in (0, 1)]
        for cp in gcps:
          cp.wait()
      for q in (0, 1):
        pltpu.sync_copy(rows.at[q, pl.ds(0, NODE_TAIL)], acc.at[lin_tail],
                        add=True)

    ne = (ESUB - s + NS - 1) // NS

    def fire_inputs_e(j, b):
      for f in range(EDGE_FIELDS):
        pltpu.async_copy(attrT_h.at[pl.ds(f * E + j * SUB, SUB)],
                         abuf.at[b, f], sem_in)
      pltpu.async_copy(ei_h.at[pl.ds(E + j * SUB, SUB)], dbuf.at[b], sem_in)

    fire_inputs_e(s, 0)

    def ebody(i, _):
      j = s + i * NS
      b = i & 1
      p0, p1 = 2 * b, 2 * b + 1
      for f in range(EDGE_FIELDS):
        pltpu.make_async_copy(attrT_h.at[pl.ds(f * E + j * SUB, SUB)],
                              abuf.at[b, f], sem_in).wait()
      pltpu.make_async_copy(ei_h.at[pl.ds(E + j * SUB, SUB)], dbuf.at[b],
                            sem_in).wait()

      @pl.when(i + 1 < ne)
      def _prefetch():
        fire_inputs_e(j + NS, 1 - b)

      @pl.when(i >= 2)
      def _drain():
        for p in (p0, p1):
          pltpu.make_async_copy(rows.at[p], acc.at[dloc.at[b]], sem_s).wait()

      for kk in range(SUB // 16):
        v = dbuf[b, pl.ds(kk * 16, 16)]
        loc = v - nbase
        ok = (v >= nbase) & (loc < HALF)
        dloc[b, pl.ds(kk * 16, 16)] = jnp.where(ok, loc, TRASH + kk * 16 + iota)

      for w in range(2):
        add = w > 0
        gcps = [pltpu.async_copy(edge_tbl.at[2 * w].at[abuf.at[b, 2 * w]],
                                 rows.at[p0], sem_g, add=add),
                pltpu.async_copy(edge_tbl.at[2 * w + 1].at[abuf.at[b, 2 * w + 1]],
                                 rows.at[p1], sem_g, add=add)]
        for cp in gcps:
          cp.wait()
      for p in (p0, p1):
        pltpu.async_copy(rows.at[p], acc.at[dloc.at[b]], sem_s, add=True)
      return 0
    lax.fori_loop(0, ne, ebody, 0)

    blast_e = (ne - 1) & 1
    for bb in (blast_e, 1 - blast_e):
      for p in (2 * bb, 2 * bb + 1):
        pltpu.make_async_copy(rows.at[p], acc.at[dloc.at[bb]], sem_s).wait()

    plsc.subcore_barrier()

    def xbody(i, _):
      j = s + i * NS
      pltpu.async_copy(acc.at[pl.ds(j * SUB, SUB)],
                       out.at[pl.ds(nbase + j * SUB, SUB)], sem_s)
      return 0
    lax.fori_loop(0, nn, xbody, 0)
    def xdrain(i, _):
      pltpu.make_async_copy(acc.at[pl.ds(s * SUB, SUB)],
                            out.at[pl.ds(nbase + s * SUB, SUB)], sem_s).wait()
      return 0
    lax.fori_loop(0, nn, xdrain, 0)

    @pl.when(s == 0)
    def _exp_tail():
      pltpu.sync_copy(acc.at[pl.ds(NODE_TAIL_BASE, NODE_TAIL)],
                      out.at[pl.ds(nbase + NODE_TAIL_BASE, NODE_TAIL)])

  return k(node_tables, edge_tables, xT, attrT, ei_flat)


MLP_BLK = 2000


def _mlp(agg, W_enc, W_dec):
  def body(a_ref, we_ref, wd_ref, o_ref):
    h = jnp.maximum(
        jnp.dot(a_ref[...], we_ref[...], preferred_element_type=jnp.float32),
        0.0)
    o_ref[...] = jnp.dot(h, wd_ref[...], preferred_element_type=jnp.float32)

  return pl.pallas_call(
      body,
      grid=(N // MLP_BLK,),
      in_specs=[
          pl.BlockSpec((MLP_BLK, D), lambda i: (i, 0)),
          pl.BlockSpec((D, H), lambda i: (0, 0)),
          pl.BlockSpec((H, D), lambda i: (0, 0)),
      ],
      out_specs=pl.BlockSpec((MLP_BLK, D), lambda i: (i, 0)),
      out_shape=jax.ShapeDtypeStruct((N, D), jnp.float32),
  )(agg, W_enc, W_dec)


def kernel(x, edge_attr, edge_index, node_tables, edge_tables, W_enc, W_dec):
  xT = jnp.transpose(x).reshape(-1)
  attrT = jnp.transpose(edge_attr).reshape(-1)
  ei_flat = edge_index.reshape(-1)
  agg = _sc_embed_aggregate(node_tables, edge_tables, xT, attrT, ei_flat)
  return _mlp(agg, W_enc, W_dec)


# speedup vs baseline: 9.4784x; 1.2294x over previous
import functools

import jax
import jax.numpy as jnp
from jax import lax
from jax.experimental import pallas as pl
from jax.experimental.pallas import tpu as pltpu
from jax.experimental.pallas import tpu_sc as plsc

N = 100000
E = 1600000
NODE_FIELDS = 8
EDGE_FIELDS = 4
D = 32
H = 64

NC = 2
NS = 16
SUB = 128

HALF = N // NC
TRASH = HALF
ACC_ROWS = 50176
NODE_FULL = HALF // SUB
NODE_TAIL = HALF - NODE_FULL * SUB
NODE_TAIL_BASE = NODE_FULL * SUB
ESUB = E // SUB
ZROWS = 16
ZSUB = ACC_ROWS // ZROWS


def _sc_embed_aggregate(node_tables, edge_tables, xT, attrT, ei_flat):
  mesh = plsc.VectorSubcoreMesh(
      core_axis_name="c", subcore_axis_name="s", num_cores=NC, num_subcores=NS)

  @functools.partial(
      pl.kernel,
      out_type=jax.ShapeDtypeStruct((N, D), jnp.float32),
      mesh=mesh,
      compiler_params=pltpu.CompilerParams(use_tc_tiling_on_sc=False),
      scratch_types=[
          pltpu.VMEM_SHARED((ACC_ROWS, D), jnp.float32),
          pltpu.VMEM((ZROWS, D), jnp.float32),
          pltpu.VMEM((2, NODE_FIELDS, SUB), jnp.int32),
          pltpu.VMEM((3, EDGE_FIELDS, SUB), jnp.int32),
          pltpu.VMEM((3, SUB), jnp.int32),
          pltpu.VMEM((3, SUB), jnp.int32),
          pltpu.VMEM((2, SUB), jnp.int32),
          pltpu.VMEM((NODE_TAIL,), jnp.int32),
          pltpu.VMEM((6, SUB, D), jnp.float32),
          pltpu.SemaphoreType.DMA,
          pltpu.SemaphoreType.DMA,
          pltpu.SemaphoreType.DMA,
          pltpu.SemaphoreType.DMA,
      ],
  )
  def k(node_tbl, edge_tbl, xT_h, attrT_h, ei_h, out, acc, zbuf, xbuf, abuf,
        dbuf, dloc, lin, lin_tail, rows, sem_in, sem_g0, sem_g1, sem_s):
    c = lax.axis_index("c")
    s = lax.axis_index("s")
    nbase = c * HALF
    iota = lax.iota(jnp.int32, 16)
    zero16 = jnp.zeros((16,), jnp.float32)

    def zrow(i, _):
      zbuf[i, pl.ds(0, 16)] = zero16
      zbuf[i, pl.ds(16, 16)] = zero16
      return 0
    lax.fori_loop(0, ZROWS, zrow, 0)

    nz = (ZSUB - s + NS - 1) // NS
    def zbody(i, _):
      j = s + i * NS
      pltpu.async_copy(zbuf, acc.at[pl.ds(j * ZROWS, ZROWS)], sem_s)
      return 0
    lax.fori_loop(0, nz, zbody, 0)
    def zdrain(i, _):
      pltpu.make_async_copy(zbuf, acc.at[pl.ds(s * ZROWS, ZROWS)], sem_s).wait()
      return 0
    lax.fori_loop(0, nz, zdrain, 0)
    plsc.subcore_barrier()

    nn = (NODE_FULL - s + NS - 1) // NS

    def fire_inputs_n(j, b):
      for f in range(NODE_FIELDS):
        pltpu.async_copy(xT_h.at[pl.ds(f * N + nbase + j * SUB, SUB)],
                         xbuf.at[b, f], sem_in)

    fire_inputs_n(s, 0)

    def nbody(i, _):
      j = s + i * NS
      b = i & 1
      p0, p1 = 2 * b, 2 * b + 1
      for f in range(NODE_FIELDS):
        pltpu.make_async_copy(xT_h.at[pl.ds(f * N + nbase + j * SUB, SUB)],
                              xbuf.at[b, f], sem_in).wait()

      @pl.when(i + 1 < nn)
      def _prefetch():
        fire_inputs_n(j + NS, 1 - b)

      lbase = j * SUB
      for kk in range(SUB // 16):
        lin[b, pl.ds(kk * 16, 16)] = lbase + kk * 16 + iota

      for w in range(NODE_FIELDS // 2):
        add = w > 0
        gcps = [pltpu.async_copy(node_tbl.at[2 * w].at[xbuf.at[b, 2 * w]],
                                 rows.at[p0], sem_g0, add=add),
                pltpu.async_copy(node_tbl.at[2 * w + 1].at[xbuf.at[b, 2 * w + 1]],
                                 rows.at[p1], sem_g0, add=add)]
        for cp in gcps:
          cp.wait()
      @pl.when(i >= 1)
      def _drain_prev():
        for p in (2 * (1 - b), 2 * (1 - b) + 1):
          pltpu.make_async_copy(rows.at[p], acc.at[lin.at[1 - b]], sem_s).wait()
      for p in (p0, p1):
        pltpu.async_copy(rows.at[p], acc.at[lin.at[b]], sem_s, add=True)
      return 0
    lax.fori_loop(0, nn, nbody, 0)

    blast_n = (nn - 1) & 1
    for p in (2 * blast_n, 2 * blast_n + 1):
      pltpu.make_async_copy(rows.at[p], acc.at[lin.at[blast_n]], sem_s).wait()

    @pl.when(s == 0)
    def _node_tail():
      for kk in range(NODE_TAIL // 16):
        lin_tail[pl.ds(kk * 16, 16)] = NODE_TAIL_BASE + kk * 16 + iota
      gbase = nbase + NODE_TAIL_BASE
      cps = [pltpu.async_copy(xT_h.at[pl.ds(f * N + gbase, NODE_TAIL)],
                              xbuf.at[0, f, pl.ds(0, NODE_TAIL)], sem_in)
             for f in range(NODE_FIELDS)]
      for cp in cps:
        cp.wait()
      for w in range(NODE_FIELDS // 2):
        gcps = [pltpu.async_copy(
            node_tbl.at[2 * w + q].at[xbuf.at[0, 2 * w + q, pl.ds(0, NODE_TAIL)]],
            rows.at[q, pl.ds(0, NODE_TAIL)], sem_g0, add=w > 0) for q in (0, 1)]
        for cp in gcps:
          cp.wait()
      for q in (0, 1):
        pltpu.sync_copy(rows.at[q, pl.ds(0, NODE_TAIL)], acc.at[lin_tail],
                        add=True)

    ne = (ESUB - s + NS - 1) // NS

    def fire_inputs_e(j, r):
      for f in range(EDGE_FIELDS):
        pltpu.async_copy(attrT_h.at[pl.ds(f * E + j * SUB, SUB)],
                         abuf.at[r, f], sem_in)
      pltpu.async_copy(ei_h.at[pl.ds(E + j * SUB, SUB)], dbuf.at[r], sem_in)

    def wait_inputs_e(j, r):
      for f in range(EDGE_FIELDS):
        pltpu.make_async_copy(attrT_h.at[pl.ds(f * E + j * SUB, SUB)],
                              abuf.at[r, f], sem_in).wait()
      pltpu.make_async_copy(ei_h.at[pl.ds(E + j * SUB, SUB)], dbuf.at[r],
                            sem_in).wait()

    def fire_wave0_e(r):
      pltpu.async_copy(edge_tbl.at[0].at[abuf.at[r, 0]], rows.at[2 * r], sem_g0)
      pltpu.async_copy(edge_tbl.at[1].at[abuf.at[r, 1]], rows.at[2 * r + 1],
                       sem_g0)

    def wait_wave0_e(r):
      for q in (0, 1):
        pltpu.make_async_copy(edge_tbl.at[q].at[abuf.at[r, q]],
                              rows.at[2 * r + q], sem_g0).wait()

    def wait_wave1_e(r):
      for q in (0, 1):
        pltpu.make_async_copy(edge_tbl.at[q].at[abuf.at[r, q]],
                              rows.at[2 * r + q], sem_g1).wait()

    def fire_wave1_e(r):
      pltpu.async_copy(edge_tbl.at[2].at[abuf.at[r, 2]], rows.at[2 * r],
                       sem_g1, add=True)
      pltpu.async_copy(edge_tbl.at[3].at[abuf.at[r, 3]], rows.at[2 * r + 1],
                       sem_g1, add=True)

    def fire_scatter_e(r):
      for q in (0, 1):
        pltpu.async_copy(rows.at[2 * r + q], acc.at[dloc.at[r]], sem_s,
                         add=True)

    def drain_scatter_e(r):
      for q in (0, 1):
        pltpu.make_async_copy(rows.at[2 * r + q], acc.at[dloc.at[r]],
                              sem_s).wait()

    fire_inputs_e(s, 0)
    fire_inputs_e(s + NS, 1)
    wait_inputs_e(s, 0)
    fire_wave0_e(0)

    def ebody(i, _):
      j = s + i * NS
      r = lax.rem(i, 3)
      r_prev = lax.rem(i + 2, 3)
      r_next = lax.rem(i + 1, 3)

      @pl.when(i + 1 < ne)
      def _wait_next_inputs():
        wait_inputs_e(j + NS, r_next)

      for kk in range(SUB // 16):
        v = dbuf[r, pl.ds(kk * 16, 16)]
        loc = v - nbase
        ok = (v >= nbase) & (loc < HALF)
        dloc[r, pl.ds(kk * 16, 16)] = jnp.where(ok, loc, TRASH + kk * 16 + iota)

      @pl.when(i >= 2)
      def _drain():
        drain_scatter_e(r_next)

      @pl.when(i >= 1)
      def _scatter_prev():
        wait_wave1_e(r_prev)
        fire_scatter_e(r_prev)

      @pl.when(i + 2 < ne)
      def _prefetch():
        fire_inputs_e(j + 2 * NS, r_prev)

      wait_wave0_e(r)
      fire_wave1_e(r)

      @pl.when(i + 1 < ne)
      def _next_wave0():
        fire_wave0_e(r_next)
      return 0
    lax.fori_loop(0, ne, ebody, 0)

    blast = ne - 1
    rl = lax.rem(blast, 3)
    drain_scatter_e(lax.rem(blast + 2, 3))
    wait_wave1_e(rl)
    fire_scatter_e(rl)
    drain_scatter_e(rl)

    plsc.subcore_barrier()

    def xbody(i, _):
      j = s + i * NS
      pltpu.async_copy(acc.at[pl.ds(j * SUB, SUB)],
                       out.at[pl.ds(nbase + j * SUB, SUB)], sem_s)
      return 0
    lax.fori_loop(0, nn, xbody, 0)
    def xdrain(i, _):
      pltpu.make_async_copy(acc.at[pl.ds(s * SUB, SUB)],
                            out.at[pl.ds(nbase + s * SUB, SUB)], sem_s).wait()
      return 0
    lax.fori_loop(0, nn, xdrain, 0)

    @pl.when(s == 0)
    def _exp_tail():
      pltpu.sync_copy(acc.at[pl.ds(NODE_TAIL_BASE, NODE_TAIL)],
                      out.at[pl.ds(nbase + NODE_TAIL_BASE, NODE_TAIL)])

  return k(node_tables, edge_tables, xT, attrT, ei_flat)


MLP_BLK = 2000


def _mlp(agg, W_enc, W_dec):
  def body(a_ref, we_ref, wd_ref, o_ref):
    h = jnp.maximum(
        jnp.dot(a_ref[...], we_ref[...], preferred_element_type=jnp.float32),
        0.0)
    o_ref[...] = jnp.dot(h, wd_ref[...], preferred_element_type=jnp.float32)

  return pl.pallas_call(
      body,
      grid=(N // MLP_BLK,),
      in_specs=[
          pl.BlockSpec((MLP_BLK, D), lambda i: (i, 0)),
          pl.BlockSpec((D, H), lambda i: (0, 0)),
          pl.BlockSpec((H, D), lambda i: (0, 0)),
      ],
      out_specs=pl.BlockSpec((MLP_BLK, D), lambda i: (i, 0)),
      out_shape=jax.ShapeDtypeStruct((N, D), jnp.float32),
  )(agg, W_enc, W_dec)


def kernel(x, edge_attr, edge_index, node_tables, edge_tables, W_enc, W_dec):
  xT = jnp.transpose(x).reshape(-1)
  attrT = jnp.transpose(edge_attr).reshape(-1)
  ei_flat = edge_index.reshape(-1)
  agg = _sc_embed_aggregate(node_tables, edge_tables, xT, attrT, ei_flat)
  return _mlp(agg, W_enc, W_dec)
